# Initial kernel scaffold; baseline (speedup 1.0000x reference)
import jax
import jax.numpy as jnp
from jax.experimental import pallas as pl


def kernel(sim):
    raise NotImplementedError("write your pallas kernel here")


# speedup vs baseline: 1.4028x; 1.4028x over previous
import functools

import jax
import jax.numpy as jnp
from jax import lax
from jax.experimental import pallas as pl
from jax.experimental.pallas import tpu as pltpu
from jax.experimental.pallas import tpu_sc as plsc

_L = 16
_K = 8
_ROWS = 64
_COLS = 8192
_CHUNKS = _COLS // _L


def _build():
    info = plsc.get_sparse_core_info()
    nc, ns = info.num_cores, info.num_subcores
    nw = nc * ns
    rows_per_w = _ROWS // nw
    mesh = plsc.VectorSubcoreMesh(core_axis_name="c", subcore_axis_name="s")

    @functools.partial(
        pl.kernel,
        mesh=mesh,
        out_type=jax.ShapeDtypeStruct((_ROWS, _L), jnp.float32),
        scratch_types=[
            pltpu.VMEM((_COLS,), jnp.float32),
            pltpu.VMEM((_COLS,), jnp.float32),
            pltpu.VMEM((_L,), jnp.float32),
            pltpu.SemaphoreType.DMA,
            pltpu.SemaphoreType.DMA,
        ],
    )
    def topk_mean(sim_hbm, out_hbm, rowa_v, rowb_v, res_v, sema, semb):
        wid = lax.axis_index("s") * nc + lax.axis_index("c")
        ra = wid * rows_per_w
        cpa = pltpu.async_copy(sim_hbm.at[ra], rowa_v, sema)
        cpb = pltpu.async_copy(sim_hbm.at[ra + 1], rowb_v, semb)
        cpa.wait()
        cpb.wait()

        neg = jnp.full((_L,), -jnp.inf, dtype=jnp.float32)
        init = (neg,) * (2 * _K)

        def body(i, carry):
            a = list(carry[:_K])
            b = list(carry[_K:])
            ca = rowa_v[pl.ds(i * _L, _L)]
            cb = rowb_v[pl.ds(i * _L, _L)]
            for j in range(_K):
                na = jnp.maximum(a[j], ca)
                ca = jnp.minimum(a[j], ca)
                a[j] = na
                nb = jnp.maximum(b[j], cb)
                cb = jnp.minimum(b[j], cb)
                b[j] = nb
            return tuple(a) + tuple(b)

        carry = lax.fori_loop(0, _CHUNKS, body, init)

        lanes = lax.iota(jnp.int32, _L)
        perms = [jnp.bitwise_xor(lanes, s) for s in (1, 2, 4, 8)]

        dnums = lax.GatherDimensionNumbers(
            offset_dims=(), collapsed_slice_dims=(0,), start_index_map=(0,))

        def shuffle(x, p):
            return lax.gather(
                x, p[:, None], dnums, slice_sizes=(1,),
                mode=lax.GatherScatterMode.PROMISE_IN_BOUNDS)

        shuffle_i32 = shuffle

        def xmax(x):
            for p in perms:
                x = jnp.maximum(x, shuffle(x, p))
            return x

        def xmin_i32(x):
            for p in perms:
                x = jnp.minimum(x, shuffle_i32(x, p))
            return x

        def top8_mean(stack):
            a = list(stack)
            s = jnp.zeros((_L,), jnp.float32)
            for _ in range(_K):
                m = xmax(a[0])
                s = s + m
                cand = jnp.where(a[0] == m, lanes, jnp.int32(_L))
                lm = lanes == xmin_i32(cand)
                for j in range(_K - 1):
                    a[j] = jnp.where(lm, a[j + 1], a[j])
                a[_K - 1] = jnp.where(lm, neg, a[_K - 1])
            return s * jnp.float32(1.0 / _K)

        res_v[...] = top8_mean(carry[:_K])
        pltpu.sync_copy(res_v, out_hbm.at[ra])

        res_v[...] = top8_mean(carry[_K:])
        pltpu.sync_copy(res_v, out_hbm.at[ra + 1])

    return topk_mean


_topk_mean = _build()


@jax.jit
def kernel(sim):
    out = _topk_mean(sim)
    return out[:, 0]
